# Initial kernel scaffold; baseline (speedup 1.0000x reference)
import jax
import jax.numpy as jnp
from jax.experimental import pallas as pl


def kernel(input_ids, weights):
    raise NotImplementedError("write your pallas kernel here")


# speedup vs baseline: 3.1838x; 3.1838x over previous
import jax
import jax.numpy as jnp
from jax.experimental import pallas as pl


def _copy_body(w_ref, o_ref):
    o_ref[...] = w_ref[...]


def kernel(input_ids, weights):
    seq_len = input_ids.shape[-1]
    num_rows, dim = weights.shape
    block_rows = 1024
    grid = (seq_len // block_rows,)
    out = pl.pallas_call(
        _copy_body,
        grid=grid,
        in_specs=[pl.BlockSpec((block_rows, dim), lambda i: (i, 0))],
        out_specs=pl.BlockSpec((block_rows, dim), lambda i: (i, 0)),
        out_shape=jax.ShapeDtypeStruct((seq_len, dim), weights.dtype),
    )(weights)
    return out
